# baseline (speedup 1.0000x reference)
import functools

import jax
import jax.numpy as jnp
from jax import lax
from jax.experimental import pallas as pl
from jax.experimental.pallas import tpu as pltpu
from jax.experimental.pallas import tpu_sc as plsc

NC = 2
NS = 16
LANES = 16
GB = 80
DC = 128


def _sc_mesh():
  return plsc.VectorSubcoreMesh(
      core_axis_name="c", subcore_axis_name="s", num_cores=NC,
      num_subcores=NS)


def _make_deg(NP, E):
  EPT = E // (NC * NS)
  NB = EPT // GB
  RP = NP // NS
  IB = 8
  assert E % (NC * NS * GB) == 0 and RP % IB == 0

  @functools.partial(
      pl.kernel,
      out_type=jax.ShapeDtypeStruct((NC * NP, DC), jnp.float32),
      mesh=_sc_mesh(),
      scratch_types=[
          pltpu.VMEM_SHARED((NP, DC), jnp.float32),
          pltpu.VMEM((NB, GB), jnp.int32),
          pltpu.VMEM((GB, DC), jnp.float32),
          pltpu.VMEM((IB, DC), jnp.float32),
      ],
  )
  def deg_k(dst_hbm, out_hbm, accum, didx, ones_b, init_b):
    cid = lax.axis_index("c")
    sid = lax.axis_index("s")
    one16 = jnp.ones((LANES,), jnp.float32)

    @pl.loop(0, GB)
    def _(i):
      for k in range(DC // LANES):
        ones_b[i, pl.ds(k * LANES, LANES)] = one16

    @pl.loop(0, IB)
    def _(i):
      for k in range(DC // LANES):
        init_b[i, pl.ds(k * LANES, LANES)] = one16

    pltpu.sync_copy(dst_hbm.at[cid * NS + sid], didx)

    @pl.loop(0, RP // IB)
    def _(j):
      pltpu.sync_copy(init_b, accum.at[pl.ds(sid * RP + j * IB, IB)])

    plsc.subcore_barrier()

    @pl.loop(0, NB)
    def _(b):
      pltpu.sync_copy(ones_b, accum.at[didx.at[b]], add=True)

    plsc.subcore_barrier()
    ob = cid * NP + sid * RP
    pltpu.sync_copy(accum.at[pl.ds(sid * RP, RP)], out_hbm.at[pl.ds(ob, RP)])

  return deg_k


def _make_agg(NP, E, C):
  EPT = E // NS
  CPC = C // NC
  RP = NP // NS
  SEGS = 10
  NBS = EPT // (GB * SEGS)
  assert EPT % (GB * SEGS) == 0 and C % NC == 0 and RP % 8 == 0
  assert SEGS % 2 == 0 and (NBS - 4) % 3 == 0 and NBS >= 7

  @functools.partial(
      pl.kernel,
      out_type=jax.ShapeDtypeStruct((C * NP, DC), jnp.float32),
      mesh=_sc_mesh(),
      scratch_types=[
          pltpu.VMEM_SHARED((NP, DC), jnp.float32),
          pltpu.VMEM((NBS, GB), jnp.int32),
          pltpu.VMEM((NBS, GB), jnp.int32),
          pltpu.VMEM((NBS, GB), jnp.int32),
          pltpu.VMEM((NBS, GB), jnp.int32),
          pltpu.VMEM((GB, DC), jnp.float32),
          pltpu.VMEM((GB, DC), jnp.float32),
          pltpu.VMEM((GB, DC), jnp.float32),
          pltpu.SemaphoreType.DMA,
          pltpu.SemaphoreType.DMA,
          pltpu.SemaphoreType.DMA,
          pltpu.SemaphoreType.DMA,
          pltpu.SemaphoreType.DMA,
          pltpu.SemaphoreType.DMA,
          pltpu.SemaphoreType.DMA,
          pltpu.SemaphoreType.DMA,
      ],
  )
  def agg_k(xs_hbm, src_hbm, dst_hbm, out_hbm, accum,
            sidxA, didxA, sidxB, didxB, gb0, gb1, gb2,
            sg0, sg1, sg2, ss0, ss1, ss2, siA, siB):
    cid = lax.axis_index("c")
    sid = lax.axis_index("s")
    gbs = (gb0, gb1, gb2)
    sgs = (sg0, sg1, sg2)
    sss = (ss0, ss1, ss2)

    def idx_load(seg, sidx, didx, sem):
      pltpu.async_copy(src_hbm.at[sid, seg], sidx, sem)
      pltpu.async_copy(dst_hbm.at[sid, seg], didx, sem)

    def idx_wait(seg, sidx, didx, sem):
      pltpu.make_async_copy(src_hbm.at[sid, seg], sidx, sem).wait()
      pltpu.make_async_copy(dst_hbm.at[sid, seg], didx, sem).wait()

    def pipeline(sidx, didx, base):
      @pl.loop(0, NBS)
      def _(b):
        for k in range(GB // LANES):
          v = sidx[b, pl.ds(k * LANES, LANES)]
          sidx[b, pl.ds(k * LANES, LANES)] = v + base

      def gath(b, k):
        pltpu.async_copy(xs_hbm.at[sidx.at[b]], gbs[k], sgs[k])

      def gath_wait(b, k):
        pltpu.make_async_copy(xs_hbm.at[sidx.at[b]], gbs[k], sgs[k]).wait()

      def scat(b, k):
        pltpu.async_copy(gbs[k], accum.at[didx.at[b]], sss[k], add=True)

      def scat_wait(b, k):
        pltpu.make_async_copy(gbs[k], accum.at[didx.at[b]], sss[k]).wait()

      for k in range(3):
        gath(k, k)

      @pl.loop(0, (NBS - 4) // 3)
      def _(i):
        b = i * 3
        for k in range(3):
          gath_wait(b + k, k)
          scat(b + k, k)
        for k in range(3):
          scat_wait(b + k, k)
          gath(b + 3 + k, k)

      for k in range(3):
        gath_wait(NBS - 4 + k, k)
        scat(NBS - 4 + k, k)
      scat_wait(NBS - 4, 0)
      gath(NBS - 1, 0)
      gath_wait(NBS - 1, 0)
      scat(NBS - 1, 0)
      scat_wait(NBS - 3, 1)
      scat_wait(NBS - 2, 2)
      scat_wait(NBS - 1, 0)

    idx_load(0, sidxA, didxA, siA)
    idx_load(1, sidxB, didxB, siB)

    @pl.loop(0, CPC)
    def _(j):
      c = cid * CPC + j
      base = c * NP
      pltpu.sync_copy(xs_hbm.at[pl.ds(base + sid * RP, RP)],
                      accum.at[pl.ds(sid * RP, RP)])
      plsc.subcore_barrier()

      @pl.loop(0, SEGS // 2)
      def _(t):
        idx_wait(2 * t, sidxA, didxA, siA)
        pipeline(sidxA, didxA, base)
        idx_load((2 * t + 2) % SEGS, sidxA, didxA, siA)
        idx_wait(2 * t + 1, sidxB, didxB, siB)
        pipeline(sidxB, didxB, base)
        idx_load((2 * t + 3) % SEGS, sidxB, didxB, siB)

      plsc.subcore_barrier()
      pltpu.sync_copy(accum.at[pl.ds(sid * RP, RP)],
                      out_hbm.at[pl.ds(base + sid * RP, RP)])

    idx_wait(0, sidxA, didxA, siA)
    idx_wait(1, sidxB, didxB, siB)

  return agg_k


def _prep_tc(deg_p, sig_p, NP, bn):
  F2 = sig_p.shape[1]
  nb = NP // bn

  def body(d0_ref, d1_ref, sig_ref, dinv_ref, s0_ref):
    deg = d0_ref[...] + d1_ref[...] - 1.0
    dinv = lax.rsqrt(deg[:, 0:1])
    dinv_ref[...] = dinv
    s0_ref[0] = dinv * sig_ref[:, pl.ds(0, DC)]
    s0_ref[1] = dinv * sig_ref[:, pl.ds(DC, DC)]

  return pl.pallas_call(
      body,
      grid=(nb,),
      in_specs=[
          pl.BlockSpec((bn, DC), lambda i: (i, 0)),
          pl.BlockSpec((bn, DC), lambda i: (nb + i, 0)),
          pl.BlockSpec((bn, F2), lambda i: (i, 0)),
      ],
      out_specs=[
          pl.BlockSpec((bn, 1), lambda i: (i, 0)),
          pl.BlockSpec((2, bn, DC), lambda i: (0, i, 0)),
      ],
      out_shape=[
          jax.ShapeDtypeStruct((NP, 1), jnp.float32),
          jax.ShapeDtypeStruct((2, NP, DC), jnp.float32),
      ],
  )(deg_p, deg_p, sig_p)


def _layer1_tc(u0, dinv, bd1, b1t, NP, C, bn):
  nb = NP // bn

  def body(u0a_ref, u0b_ref, dinv_ref, bd1_ref, b1_ref, out_ref):
    t = (jnp.dot(u0a_ref[...], bd1_ref[pl.ds(0, DC), :],
                 preferred_element_type=jnp.float32) +
         jnp.dot(u0b_ref[...], bd1_ref[pl.ds(DC, DC), :],
                 preferred_element_type=jnp.float32))
    dinv = dinv_ref[...]
    out_ref[...] = dinv * jnp.maximum(dinv * t + b1_ref[...], 0.0)

  return pl.pallas_call(
      body,
      grid=(nb, C),
      in_specs=[
          pl.BlockSpec((bn, DC), lambda i, c: (i, 0)),
          pl.BlockSpec((bn, DC), lambda i, c: (nb + i, 0)),
          pl.BlockSpec((bn, 1), lambda i, c: (i, 0)),
          pl.BlockSpec((2 * DC, DC), lambda i, c: (0, c)),
          pl.BlockSpec((1, DC), lambda i, c: (0, c)),
      ],
      out_specs=pl.BlockSpec((bn, DC), lambda i, c: (c * nb + i, 0)),
      out_shape=jax.ShapeDtypeStruct((C * NP, DC), jnp.float32),
  )(u0, u0, dinv, bd1, b1t)


def _layer23_tc(u1, dinv, W2, b2, W3, NP, C, G, bn):
  H = W2.shape[1]
  K = H // DC
  O = W3.shape[1]
  nb = NP // bn
  F2 = 2 * DC

  def body(u1_ref, dinv_ref, w2_ref, b2_ref, w3_ref, out_ref):
    dinv = dinv_ref[...]
    out_ref[...] = jnp.zeros((bn, F2), jnp.float32)
    for g in range(G):
      acc = jnp.zeros((bn, H), jnp.float32)
      for k in range(K):
        acc = acc + jnp.dot(u1_ref[g * K + k], w2_ref[pl.ds(k * DC, DC), :],
                            preferred_element_type=jnp.float32)
      x2g = jnp.maximum(dinv * acc + b2_ref[...], 0.0)
      y3g = jnp.dot(x2g, w3_ref[...], preferred_element_type=jnp.float32)
      out_ref[:, pl.ds(g * O, O)] = dinv * y3g

  return pl.pallas_call(
      body,
      grid=(nb,),
      in_specs=[
          pl.BlockSpec((C, bn, DC), lambda i: (0, i, 0)),
          pl.BlockSpec((bn, 1), lambda i: (i, 0)),
          pl.BlockSpec((H, H), lambda i: (0, 0)),
          pl.BlockSpec((1, H), lambda i: (0, 0)),
          pl.BlockSpec((H, O), lambda i: (0, 0)),
      ],
      out_specs=pl.BlockSpec((bn, F2), lambda i: (i, 0)),
      out_shape=jax.ShapeDtypeStruct((NP, F2), jnp.float32),
  )(u1, dinv, W2, b2, W3)


def _out_tc(u2, dinv, b3t, NP, bn):
  nb = NP // bn

  def body(u2_ref, dinv_ref, b3_ref, out_ref):
    out_ref[...] = jnp.maximum(dinv_ref[...] * u2_ref[...] + b3_ref[...], 0.0)

  return pl.pallas_call(
      body,
      grid=(nb, 2),
      in_specs=[
          pl.BlockSpec((bn, DC), lambda i, c: (c * nb + i, 0)),
          pl.BlockSpec((bn, 1), lambda i, c: (i, 0)),
          pl.BlockSpec((1, DC), lambda i, c: (0, c)),
      ],
      out_specs=pl.BlockSpec((bn, DC), lambda i, c: (i, c)),
      out_shape=jax.ShapeDtypeStruct((NP, 2 * DC), jnp.float32),
  )(u2, dinv, b3t)


def kernel(signal, edge_index, W1, b1, W2, b2, W3, b3):
  NN, F = signal.shape
  E = edge_index.shape[1]
  IN = W1.shape[0]
  H = W1.shape[1]
  G = F // IN
  C = (G * H) // DC
  F2 = 2 * DC
  NP = 10240
  assert NN <= NP and E % (NC * NS * GB) == 0 and F <= F2

  src1 = edge_index[0].reshape(NC * NS, E // (NC * NS * GB), GB)
  dst1 = edge_index[1].reshape(NC * NS, E // (NC * NS * GB), GB)
  SEGS = 10
  src2 = edge_index[0].reshape(NS, SEGS, E // (NS * SEGS * GB), GB)
  dst2 = edge_index[1].reshape(NS, SEGS, E // (NS * SEGS * GB), GB)

  sig_p = jnp.pad(signal, ((0, NP - NN), (0, F2 - F)))

  bd1 = jnp.pad(jnp.kron(jnp.eye(G, dtype=W1.dtype), W1), ((0, F2 - F), (0, 0)))
  b1t = jnp.tile(b1, (G,))[None, :]
  b2r = b2[None, :]
  b3t = jnp.pad(jnp.tile(b3, (G,)), (0, F2 - F))[None, :]

  deg_p = _make_deg(NP, E)(dst1)
  dinv, s0 = _prep_tc(deg_p, sig_p, NP, bn=1024)

  u0 = _make_agg(NP, E, 2)(s0.reshape(2 * NP, DC), src2, dst2)
  x1s = _layer1_tc(u0, dinv, bd1, b1t, NP, C, bn=1024)

  u1 = _make_agg(NP, E, C)(x1s, src2, dst2)

  s2 = _layer23_tc(u1.reshape(C, NP, DC), dinv, W2, b2r, W3, NP, C, G, bn=256)

  s2c = s2.reshape(NP, 2, DC).transpose(1, 0, 2).reshape(2 * NP, DC)
  u2 = _make_agg(NP, E, 2)(s2c, src2, dst2)
  out = _out_tc(u2, dinv, b3t, NP, bn=1024)
  return out[:NN, :F]


# speedup vs baseline: 1.0545x; 1.0545x over previous
import functools

import jax
import jax.numpy as jnp
from jax import lax
from jax.experimental import pallas as pl
from jax.experimental.pallas import tpu as pltpu
from jax.experimental.pallas import tpu_sc as plsc

NC = 2
NS = 16
LANES = 16
GB = 80
DC = 128


def _sc_mesh():
  return plsc.VectorSubcoreMesh(
      core_axis_name="c", subcore_axis_name="s", num_cores=NC,
      num_subcores=NS)


def _make_deg(NP, E):
  EPT = E // (NC * NS)
  NB = EPT // GB
  RP = NP // NS
  IB = 128
  assert E % (NC * NS * GB) == 0 and RP % IB == 0

  @functools.partial(
      pl.kernel,
      out_type=jax.ShapeDtypeStruct((NC * NP, DC), jnp.float32),
      mesh=_sc_mesh(),
      scratch_types=[
          pltpu.VMEM_SHARED((NP, DC), jnp.float32),
          pltpu.VMEM((NB, GB), jnp.int32),
          pltpu.VMEM((GB, DC), jnp.float32),
          pltpu.VMEM((IB, DC), jnp.float32),
      ],
  )
  def deg_k(dst_hbm, out_hbm, accum, didx, ones_b, init_b):
    cid = lax.axis_index("c")
    sid = lax.axis_index("s")
    one16 = jnp.ones((LANES,), jnp.float32)

    @pl.loop(0, GB)
    def _(i):
      for k in range(DC // LANES):
        ones_b[i, pl.ds(k * LANES, LANES)] = one16

    @pl.loop(0, IB)
    def _(i):
      for k in range(DC // LANES):
        init_b[i, pl.ds(k * LANES, LANES)] = one16

    pltpu.sync_copy(dst_hbm.at[cid * NS + sid], didx)

    @pl.loop(0, RP // IB)
    def _(j):
      pltpu.sync_copy(init_b, accum.at[pl.ds(sid * RP + j * IB, IB)])

    plsc.subcore_barrier()

    @pl.loop(0, NB)
    def _(b):
      pltpu.sync_copy(ones_b, accum.at[didx.at[b]], add=True)

    plsc.subcore_barrier()
    ob = cid * NP + sid * RP
    pltpu.sync_copy(accum.at[pl.ds(sid * RP, RP)], out_hbm.at[pl.ds(ob, RP)])

  return deg_k


def _make_agg(NP, E, C):
  EPT = E // NS
  CPC = C // NC
  RP = NP // NS
  SEGS = 5
  NBS = EPT // (GB * SEGS)
  assert EPT % (GB * SEGS) == 0 and C % NC == 0 and RP % 8 == 0
  assert NBS % 2 == 0

  @functools.partial(
      pl.kernel,
      out_type=jax.ShapeDtypeStruct((C * NP, DC), jnp.float32),
      mesh=_sc_mesh(),
      scratch_types=[
          pltpu.VMEM_SHARED((NP, DC), jnp.float32),
          pltpu.VMEM((NBS, GB), jnp.int32),
          pltpu.VMEM((NBS, GB), jnp.int32),
          pltpu.VMEM((GB, DC), jnp.float32),
          pltpu.VMEM((GB, DC), jnp.float32),
          pltpu.SemaphoreType.DMA,
          pltpu.SemaphoreType.DMA,
      ],
  )
  def agg_k(xs_hbm, src_hbm, dst_hbm, out_hbm, accum, sidx, didx, gb0, gb1,
            sem0, sem1):
    cid = lax.axis_index("c")
    sid = lax.axis_index("s")

    @pl.loop(0, CPC)
    def _(j):
      c = cid * CPC + j
      base = c * NP
      pltpu.sync_copy(xs_hbm.at[pl.ds(base + sid * RP, RP)],
                      accum.at[pl.ds(sid * RP, RP)])
      plsc.subcore_barrier()

      @pl.loop(0, SEGS)
      def _(seg):
        pltpu.sync_copy(src_hbm.at[sid, seg], sidx)
        pltpu.sync_copy(dst_hbm.at[sid, seg], didx)

        @pl.loop(0, NBS)
        def _(b):
          for k in range(GB // LANES):
            v = sidx[b, pl.ds(k * LANES, LANES)]
            sidx[b, pl.ds(k * LANES, LANES)] = v + base

        pltpu.async_copy(xs_hbm.at[sidx.at[0]], gb0, sem0)

        @pl.loop(0, NBS, step=2)
        def _(b):
          pltpu.async_copy(xs_hbm.at[sidx.at[b + 1]], gb1, sem1)
          pltpu.make_async_copy(xs_hbm.at[sidx.at[b]], gb0, sem0).wait()
          pltpu.sync_copy(gb0, accum.at[didx.at[b]], add=True)

          @pl.when(b + 2 < NBS)
          def _():
            pltpu.async_copy(xs_hbm.at[sidx.at[b + 2]], gb0, sem0)

          pltpu.make_async_copy(xs_hbm.at[sidx.at[b + 1]], gb1, sem1).wait()
          pltpu.sync_copy(gb1, accum.at[didx.at[b + 1]], add=True)

      plsc.subcore_barrier()
      pltpu.sync_copy(accum.at[pl.ds(sid * RP, RP)],
                      out_hbm.at[pl.ds(base + sid * RP, RP)])
      plsc.subcore_barrier()

  return agg_k


def _prep_tc(deg_p, sig_p, NP, bn):
  F2 = sig_p.shape[1]
  nb = NP // bn

  def body(d0_ref, d1_ref, sig_ref, dinv_ref, s0_ref):
    deg = d0_ref[...] + d1_ref[...] - 1.0
    dinv = lax.rsqrt(deg[:, 0:1])
    dinv_ref[...] = dinv
    s0_ref[0] = dinv * sig_ref[:, pl.ds(0, DC)]
    s0_ref[1] = dinv * sig_ref[:, pl.ds(DC, DC)]

  return pl.pallas_call(
      body,
      grid=(nb,),
      in_specs=[
          pl.BlockSpec((bn, DC), lambda i: (i, 0)),
          pl.BlockSpec((bn, DC), lambda i: (nb + i, 0)),
          pl.BlockSpec((bn, F2), lambda i: (i, 0)),
      ],
      out_specs=[
          pl.BlockSpec((bn, 1), lambda i: (i, 0)),
          pl.BlockSpec((2, bn, DC), lambda i: (0, i, 0)),
      ],
      out_shape=[
          jax.ShapeDtypeStruct((NP, 1), jnp.float32),
          jax.ShapeDtypeStruct((2, NP, DC), jnp.float32),
      ],
  )(deg_p, deg_p, sig_p)


def _layer1_tc(u0, dinv, bd1, b1t, NP, C, bn):
  nb = NP // bn

  def body(u0a_ref, u0b_ref, dinv_ref, bd1_ref, b1_ref, out_ref):
    t = (jnp.dot(u0a_ref[...], bd1_ref[pl.ds(0, DC), :],
                 preferred_element_type=jnp.float32) +
         jnp.dot(u0b_ref[...], bd1_ref[pl.ds(DC, DC), :],
                 preferred_element_type=jnp.float32))
    dinv = dinv_ref[...]
    out_ref[...] = dinv * jnp.maximum(dinv * t + b1_ref[...], 0.0)

  return pl.pallas_call(
      body,
      grid=(nb, C),
      in_specs=[
          pl.BlockSpec((bn, DC), lambda i, c: (i, 0)),
          pl.BlockSpec((bn, DC), lambda i, c: (nb + i, 0)),
          pl.BlockSpec((bn, 1), lambda i, c: (i, 0)),
          pl.BlockSpec((2 * DC, DC), lambda i, c: (0, c)),
          pl.BlockSpec((1, DC), lambda i, c: (0, c)),
      ],
      out_specs=pl.BlockSpec((bn, DC), lambda i, c: (c * nb + i, 0)),
      out_shape=jax.ShapeDtypeStruct((C * NP, DC), jnp.float32),
  )(u0, u0, dinv, bd1, b1t)


def _layer23h_tc(u1h, dinv, W2, b2, W3, NP, GH, bn):
  H = W2.shape[1]
  K = H // DC
  O = W3.shape[1]
  CH = K * GH
  nb = NP // bn

  def body(u1_ref, dinv_ref, w2_ref, b2_ref, w3_ref, out_ref):
    dinv = dinv_ref[...]
    out_ref[...] = jnp.zeros((bn, DC), jnp.float32)
    for g in range(GH):
      acc = jnp.zeros((bn, H), jnp.float32)
      for k in range(K):
        acc = acc + jnp.dot(u1_ref[g * K + k], w2_ref[pl.ds(k * DC, DC), :],
                            preferred_element_type=jnp.float32)
      x2g = jnp.maximum(dinv * acc + b2_ref[...], 0.0)
      y3g = jnp.dot(x2g, w3_ref[...], preferred_element_type=jnp.float32)
      out_ref[:, pl.ds(g * O, O)] = dinv * y3g

  return pl.pallas_call(
      body,
      grid=(nb,),
      in_specs=[
          pl.BlockSpec((CH, bn, DC), lambda i: (0, i, 0)),
          pl.BlockSpec((bn, 1), lambda i: (i, 0)),
          pl.BlockSpec((H, H), lambda i: (0, 0)),
          pl.BlockSpec((1, H), lambda i: (0, 0)),
          pl.BlockSpec((H, O), lambda i: (0, 0)),
      ],
      out_specs=pl.BlockSpec((bn, DC), lambda i: (i, 0)),
      out_shape=jax.ShapeDtypeStruct((NP, DC), jnp.float32),
  )(u1h, dinv, W2, b2, W3)


def _out_tc(u2, dinv, b3t, NP, bn):
  nb = NP // bn

  def body(u2_ref, dinv_ref, b3_ref, out_ref):
    out_ref[...] = jnp.maximum(dinv_ref[...] * u2_ref[...] + b3_ref[...], 0.0)

  return pl.pallas_call(
      body,
      grid=(nb, 2),
      in_specs=[
          pl.BlockSpec((bn, DC), lambda i, c: (c * nb + i, 0)),
          pl.BlockSpec((bn, 1), lambda i, c: (i, 0)),
          pl.BlockSpec((1, DC), lambda i, c: (0, c)),
      ],
      out_specs=pl.BlockSpec((bn, DC), lambda i, c: (i, c)),
      out_shape=jax.ShapeDtypeStruct((NP, 2 * DC), jnp.float32),
  )(u2, dinv, b3t)


def kernel(signal, edge_index, W1, b1, W2, b2, W3, b3):
  NN, F = signal.shape
  E = edge_index.shape[1]
  IN = W1.shape[0]
  H = W1.shape[1]
  G = F // IN
  GH = G // 2
  CH = (GH * H) // DC
  F2 = 2 * DC
  OH = GH * IN
  NP = 10240
  assert NN <= NP and E % (NC * NS * GB) == 0 and F <= F2

  SEGS = 5
  src1 = edge_index[0].reshape(NC * NS, E // (NC * NS * GB), GB)
  dst1 = edge_index[1].reshape(NC * NS, E // (NC * NS * GB), GB)
  src2 = edge_index[0].reshape(NS, SEGS, E // (NS * SEGS * GB), GB)
  dst2 = edge_index[1].reshape(NS, SEGS, E // (NS * SEGS * GB), GB)

  sig_p = jnp.pad(signal, ((0, NP - NN), (0, F2 - F)))

  bd1 = jnp.pad(jnp.kron(jnp.eye(G, dtype=W1.dtype), W1), ((0, F2 - F), (0, 0)))
  b1t = jnp.tile(b1, (G,))[None, :]
  b2r = b2[None, :]
  b3h = jnp.pad(jnp.tile(b3, (GH,)), (0, DC - OH))
  b3t = jnp.concatenate([b3h, b3h])[None, :]

  deg_p = _make_deg(NP, E)(dst1)
  dinv, s0 = _prep_tc(deg_p, sig_p, NP, bn=1024)

  u0 = _make_agg(NP, E, 2)(s0.reshape(2 * NP, DC), src2, dst2)
  x1a = _layer1_tc(u0, dinv, bd1[:, :CH * DC], b1t[:, :CH * DC], NP, CH,
                   bn=1024)
  u1a = _make_agg(NP, E, CH)(x1a, src2, dst2)
  x1b = _layer1_tc(u0, dinv, bd1[:, CH * DC:], b1t[:, CH * DC:], NP, CH,
                   bn=1024)
  u1b = _make_agg(NP, E, CH)(x1b, src2, dst2)

  s2a = _layer23h_tc(u1a.reshape(CH, NP, DC), dinv, W2, b2r, W3, NP, GH,
                     bn=256)
  s2b = _layer23h_tc(u1b.reshape(CH, NP, DC), dinv, W2, b2r, W3, NP, GH,
                     bn=256)

  s2c = jnp.concatenate([s2a, s2b], axis=0)
  u2 = _make_agg(NP, E, 2)(s2c, src2, dst2)
  out = _out_tc(u2, dinv, b3t, NP, bn=1024)
  return jnp.concatenate([out[:NN, :OH], out[:NN, DC:DC + OH]], axis=1)


# speedup vs baseline: 1.0634x; 1.0084x over previous
import functools

import jax
import jax.numpy as jnp
from jax import lax
from jax.experimental import pallas as pl
from jax.experimental.pallas import tpu as pltpu
from jax.experimental.pallas import tpu_sc as plsc

NC = 2
NS = 16
LANES = 16
GB = 80
DC = 128


def _sc_mesh():
  return plsc.VectorSubcoreMesh(
      core_axis_name="c", subcore_axis_name="s", num_cores=NC,
      num_subcores=NS)


def _make_deg(NP, E):
  EPT = E // (NC * NS)
  NB = EPT // GB
  RP = NP // NS
  IB = 128
  assert E % (NC * NS * GB) == 0 and RP % IB == 0

  @functools.partial(
      pl.kernel,
      out_type=jax.ShapeDtypeStruct((NC * NP, DC), jnp.float32),
      mesh=_sc_mesh(),
      scratch_types=[
          pltpu.VMEM_SHARED((NP, DC), jnp.float32),
          pltpu.VMEM((NB, GB), jnp.int32),
          pltpu.VMEM((GB, DC), jnp.float32),
          pltpu.VMEM((IB, DC), jnp.float32),
      ],
  )
  def deg_k(dst_hbm, out_hbm, accum, didx, ones_b, init_b):
    cid = lax.axis_index("c")
    sid = lax.axis_index("s")
    one16 = jnp.ones((LANES,), jnp.float32)

    @pl.loop(0, GB)
    def _(i):
      for k in range(DC // LANES):
        ones_b[i, pl.ds(k * LANES, LANES)] = one16

    @pl.loop(0, IB)
    def _(i):
      for k in range(DC // LANES):
        init_b[i, pl.ds(k * LANES, LANES)] = one16

    pltpu.sync_copy(dst_hbm.at[cid * NS + sid], didx)

    @pl.loop(0, RP // IB)
    def _(j):
      pltpu.sync_copy(init_b, accum.at[pl.ds(sid * RP + j * IB, IB)])

    plsc.subcore_barrier()

    @pl.loop(0, NB)
    def _(b):
      pltpu.sync_copy(ones_b, accum.at[didx.at[b]], add=True)

    plsc.subcore_barrier()
    ob = cid * NP + sid * RP
    pltpu.sync_copy(accum.at[pl.ds(sid * RP, RP)], out_hbm.at[pl.ds(ob, RP)])

  return deg_k


def _make_agg(NP, E, C):
  EPT = E // NS
  CPC = C // NC
  RP = NP // NS
  SEGS = 5
  NBS = EPT // (GB * SEGS)
  assert EPT % (GB * SEGS) == 0 and C % NC == 0 and RP % 8 == 0
  assert NBS % 2 == 0

  @functools.partial(
      pl.kernel,
      out_type=jax.ShapeDtypeStruct((C * NP, DC), jnp.float32),
      mesh=_sc_mesh(),
      scratch_types=[
          pltpu.VMEM_SHARED((NP, DC), jnp.float32),
          pltpu.VMEM((NBS, GB), jnp.int32),
          pltpu.VMEM((NBS, GB), jnp.int32),
          pltpu.VMEM((GB, DC), jnp.float32),
          pltpu.VMEM((GB, DC), jnp.float32),
          pltpu.SemaphoreType.DMA,
          pltpu.SemaphoreType.DMA,
      ],
  )
  def agg_k(xs_hbm, src_hbm, dst_hbm, out_hbm, accum, sidx, didx, gb0, gb1,
            sem0, sem1):
    cid = lax.axis_index("c")
    sid = lax.axis_index("s")

    @pl.loop(0, CPC)
    def _(j):
      c = cid * CPC + j
      base = c * NP
      pltpu.sync_copy(xs_hbm.at[pl.ds(base + sid * RP, RP)],
                      accum.at[pl.ds(sid * RP, RP)])
      plsc.subcore_barrier()

      @pl.loop(0, SEGS)
      def _(seg):
        pltpu.sync_copy(src_hbm.at[sid, seg], sidx)
        pltpu.sync_copy(dst_hbm.at[sid, seg], didx)

        @pl.loop(0, NBS)
        def _(b):
          for k in range(GB // LANES):
            v = sidx[b, pl.ds(k * LANES, LANES)]
            sidx[b, pl.ds(k * LANES, LANES)] = v + base

        pltpu.async_copy(xs_hbm.at[sidx.at[0]], gb0, sem0)

        @pl.loop(0, NBS, step=2)
        def _(b):
          pltpu.async_copy(xs_hbm.at[sidx.at[b + 1]], gb1, sem1)
          pltpu.make_async_copy(xs_hbm.at[sidx.at[b]], gb0, sem0).wait()
          pltpu.sync_copy(gb0, accum.at[didx.at[b]], add=True)

          @pl.when(b + 2 < NBS)
          def _():
            pltpu.async_copy(xs_hbm.at[sidx.at[b + 2]], gb0, sem0)

          pltpu.make_async_copy(xs_hbm.at[sidx.at[b + 1]], gb1, sem1).wait()
          pltpu.sync_copy(gb1, accum.at[didx.at[b + 1]], add=True)

      plsc.subcore_barrier()
      pltpu.sync_copy(accum.at[pl.ds(sid * RP, RP)],
                      out_hbm.at[pl.ds(base + sid * RP, RP)])
      plsc.subcore_barrier()

  return agg_k


def _make_agg_es(NP, E):
  EPT = E // (NC * NS)
  RP = NP // NS
  SEGS = 5
  NBS = EPT // (GB * SEGS)
  assert EPT % (GB * SEGS) == 0 and RP % 8 == 0

  @functools.partial(
      pl.kernel,
      out_type=jax.ShapeDtypeStruct((NC * NP, DC), jnp.float32),
      mesh=_sc_mesh(),
      scratch_types=[
          pltpu.VMEM_SHARED((NP, DC), jnp.float32),
          pltpu.VMEM((NBS, GB), jnp.int32),
          pltpu.VMEM((NBS, GB), jnp.int32),
          pltpu.VMEM((GB, DC), jnp.float32),
          pltpu.VMEM((GB, DC), jnp.float32),
          pltpu.SemaphoreType.DMA,
          pltpu.SemaphoreType.DMA,
      ],
  )
  def agg_k(xs_hbm, src_hbm, dst_hbm, out_hbm, accum, sidx, didx, gb0, gb1,
            sem0, sem1):
    cid = lax.axis_index("c")
    sid = lax.axis_index("s")
    w = cid * NS + sid

    pltpu.sync_copy(xs_hbm.at[pl.ds(sid * RP, RP)],
                    accum.at[pl.ds(sid * RP, RP)])
    plsc.subcore_barrier()

    @pl.loop(0, SEGS)
    def _(seg):
      pltpu.sync_copy(src_hbm.at[w, seg], sidx)
      pltpu.sync_copy(dst_hbm.at[w, seg], didx)
      pltpu.async_copy(xs_hbm.at[sidx.at[0]], gb0, sem0)

      @pl.loop(0, NBS, step=2)
      def _(b):
        @pl.when(b + 1 < NBS)
        def _():
          pltpu.async_copy(xs_hbm.at[sidx.at[b + 1]], gb1, sem1)

        pltpu.make_async_copy(xs_hbm.at[sidx.at[b]], gb0, sem0).wait()
        pltpu.sync_copy(gb0, accum.at[didx.at[b]], add=True)

        @pl.when(b + 2 < NBS)
        def _():
          pltpu.async_copy(xs_hbm.at[sidx.at[b + 2]], gb0, sem0)

        @pl.when(b + 1 < NBS)
        def _():
          pltpu.make_async_copy(xs_hbm.at[sidx.at[b + 1]], gb1, sem1).wait()
          pltpu.sync_copy(gb1, accum.at[didx.at[b + 1]], add=True)

    plsc.subcore_barrier()
    ob = cid * NP + sid * RP
    pltpu.sync_copy(accum.at[pl.ds(sid * RP, RP)], out_hbm.at[pl.ds(ob, RP)])

  return agg_k


def _prep_tc(deg_p, sig_p, NP, bn):
  F2 = sig_p.shape[1]
  nb = NP // bn

  def body(d0_ref, d1_ref, sig_ref, dinv_ref, s0_ref):
    deg = d0_ref[...] + d1_ref[...] - 1.0
    dinv = lax.rsqrt(deg[:, 0:1])
    dinv_ref[...] = dinv
    s0_ref[0] = dinv * sig_ref[:, pl.ds(0, DC)]
    s0_ref[1] = dinv * sig_ref[:, pl.ds(DC, DC)]

  return pl.pallas_call(
      body,
      grid=(nb,),
      in_specs=[
          pl.BlockSpec((bn, DC), lambda i: (i, 0)),
          pl.BlockSpec((bn, DC), lambda i: (nb + i, 0)),
          pl.BlockSpec((bn, F2), lambda i: (i, 0)),
      ],
      out_specs=[
          pl.BlockSpec((bn, 1), lambda i: (i, 0)),
          pl.BlockSpec((2, bn, DC), lambda i: (0, i, 0)),
      ],
      out_shape=[
          jax.ShapeDtypeStruct((NP, 1), jnp.float32),
          jax.ShapeDtypeStruct((2, NP, DC), jnp.float32),
      ],
  )(deg_p, deg_p, sig_p)


def _layer1_tc(u0, dinv, bd1, b1t, NP, C, bn):
  nb = NP // bn

  def body(u0a_ref, u0b_ref, dinv_ref, bd1_ref, b1_ref, out_ref):
    t = (jnp.dot(u0a_ref[...], bd1_ref[pl.ds(0, DC), :],
                 preferred_element_type=jnp.float32) +
         jnp.dot(u0b_ref[...], bd1_ref[pl.ds(DC, DC), :],
                 preferred_element_type=jnp.float32))
    dinv = dinv_ref[...]
    out_ref[...] = dinv * jnp.maximum(dinv * t + b1_ref[...], 0.0)

  return pl.pallas_call(
      body,
      grid=(nb, C),
      in_specs=[
          pl.BlockSpec((bn, DC), lambda i, c: (i, 0)),
          pl.BlockSpec((bn, DC), lambda i, c: (nb + i, 0)),
          pl.BlockSpec((bn, 1), lambda i, c: (i, 0)),
          pl.BlockSpec((2 * DC, DC), lambda i, c: (0, c)),
          pl.BlockSpec((1, DC), lambda i, c: (0, c)),
      ],
      out_specs=pl.BlockSpec((bn, DC), lambda i, c: (c * nb + i, 0)),
      out_shape=jax.ShapeDtypeStruct((C * NP, DC), jnp.float32),
  )(u0, u0, dinv, bd1, b1t)


def _layer23h_tc(u1h, dinv, W2, b2, W3, NP, GH, bn):
  H = W2.shape[1]
  K = H // DC
  O = W3.shape[1]
  CH = K * GH
  nb = NP // bn

  def body(u1_ref, dinv_ref, w2_ref, b2_ref, w3_ref, out_ref):
    dinv = dinv_ref[...]
    out_ref[...] = jnp.zeros((bn, DC), jnp.float32)
    for g in range(GH):
      acc = jnp.zeros((bn, H), jnp.float32)
      for k in range(K):
        acc = acc + jnp.dot(u1_ref[g * K + k], w2_ref[pl.ds(k * DC, DC), :],
                            preferred_element_type=jnp.float32)
      x2g = jnp.maximum(dinv * acc + b2_ref[...], 0.0)
      y3g = jnp.dot(x2g, w3_ref[...], preferred_element_type=jnp.float32)
      out_ref[:, pl.ds(g * O, O)] = dinv * y3g

  return pl.pallas_call(
      body,
      grid=(nb,),
      in_specs=[
          pl.BlockSpec((CH, bn, DC), lambda i: (0, i, 0)),
          pl.BlockSpec((bn, 1), lambda i: (i, 0)),
          pl.BlockSpec((H, H), lambda i: (0, 0)),
          pl.BlockSpec((1, H), lambda i: (0, 0)),
          pl.BlockSpec((H, O), lambda i: (0, 0)),
      ],
      out_specs=pl.BlockSpec((bn, DC), lambda i: (i, 0)),
      out_shape=jax.ShapeDtypeStruct((NP, DC), jnp.float32),
  )(u1h, dinv, W2, b2, W3)


def _out_tc(p, s2c, dinv, b3t, NP, bn):
  nb = NP // bn

  def body(p0_ref, p1_ref, s2_ref, dinv_ref, b3_ref, out_ref):
    u = p0_ref[...] + p1_ref[...] - s2_ref[...]
    out_ref[...] = jnp.maximum(dinv_ref[...] * u + b3_ref[...], 0.0)

  return pl.pallas_call(
      body,
      grid=(nb, 2),
      in_specs=[
          pl.BlockSpec((bn, DC), lambda i, c: (2 * c * nb + i, 0)),
          pl.BlockSpec((bn, DC), lambda i, c: ((2 * c + 1) * nb + i, 0)),
          pl.BlockSpec((bn, DC), lambda i, c: (c * nb + i, 0)),
          pl.BlockSpec((bn, 1), lambda i, c: (i, 0)),
          pl.BlockSpec((1, DC), lambda i, c: (0, c)),
      ],
      out_specs=pl.BlockSpec((bn, DC), lambda i, c: (i, c)),
      out_shape=jax.ShapeDtypeStruct((NP, 2 * DC), jnp.float32),
  )(p, p, s2c, dinv, b3t)


def kernel(signal, edge_index, W1, b1, W2, b2, W3, b3):
  NN, F = signal.shape
  E = edge_index.shape[1]
  IN = W1.shape[0]
  H = W1.shape[1]
  G = F // IN
  GH = G // 2
  CH = (GH * H) // DC
  F2 = 2 * DC
  OH = GH * IN
  NP = 10240
  assert NN <= NP and E % (NC * NS * GB) == 0 and F <= F2

  SEGS = 5
  src1 = edge_index[0].reshape(NC * NS, E // (NC * NS * GB), GB)
  dst1 = edge_index[1].reshape(NC * NS, E // (NC * NS * GB), GB)
  src3 = edge_index[0].reshape(NC * NS, SEGS, E // (NC * NS * SEGS * GB), GB)
  dst3 = edge_index[1].reshape(NC * NS, SEGS, E // (NC * NS * SEGS * GB), GB)
  src2 = edge_index[0].reshape(NS, SEGS, E // (NS * SEGS * GB), GB)
  dst2 = edge_index[1].reshape(NS, SEGS, E // (NS * SEGS * GB), GB)

  sig_p = jnp.pad(signal, ((0, NP - NN), (0, F2 - F)))

  bd1 = jnp.pad(jnp.kron(jnp.eye(G, dtype=W1.dtype), W1), ((0, F2 - F), (0, 0)))
  b1t = jnp.tile(b1, (G,))[None, :]
  b2r = b2[None, :]
  b3h = jnp.pad(jnp.tile(b3, (GH,)), (0, DC - OH))
  b3t = jnp.concatenate([b3h, b3h])[None, :]

  deg_p = _make_deg(NP, E)(dst1)
  dinv, s0 = _prep_tc(deg_p, sig_p, NP, bn=1024)

  u0 = _make_agg(NP, E, 2)(s0.reshape(2 * NP, DC), src2, dst2)
  x1a = _layer1_tc(u0, dinv, bd1[:, :CH * DC], b1t[:, :CH * DC], NP, CH,
                   bn=1024)
  u1a = _make_agg(NP, E, CH)(x1a, src2, dst2)
  x1b = _layer1_tc(u0, dinv, bd1[:, CH * DC:], b1t[:, CH * DC:], NP, CH,
                   bn=1024)
  s2a = _layer23h_tc(u1a.reshape(CH, NP, DC), dinv, W2, b2r, W3, NP, GH,
                     bn=256)
  u1b = _make_agg(NP, E, CH)(x1b, src2, dst2)
  s2b = _layer23h_tc(u1b.reshape(CH, NP, DC), dinv, W2, b2r, W3, NP, GH,
                     bn=256)

  pa = _make_agg_es(NP, E)(s2a, src3, dst3)
  pb = _make_agg_es(NP, E)(s2b, src3, dst3)
  p = jnp.concatenate([pa, pb], axis=0)
  s2c = jnp.concatenate([s2a, s2b], axis=0)
  out = _out_tc(p, s2c, dinv, b3t, NP, bn=1024)
  return jnp.concatenate([out[:NN, :OH], out[:NN, DC:DC + OH]], axis=1)
